# baseline (speedup 1.0000x reference)
import functools

import jax
import jax.numpy as jnp
from jax import lax
from jax.experimental import pallas as pl
from jax.experimental.pallas import tpu as pltpu
from jax.experimental.pallas import tpu_sc as plsc

N = 10000
E = 320000
D_IN = 128
D_HID = 16
D_OUT = 128

NC = 2
NS = 16
NW = NC * NS
B = 125
CHP = 80
N_PAD = 10240
RPS = N_PAD // NS
NR = N_PAD // 8
K = 8
G = CHP // K
G_DEG = G

_mesh = functools.partial(
    pl.kernel,
    mesh=plsc.VectorSubcoreMesh(core_axis_name="c", subcore_axis_name="s"),
    compiler_params=pltpu.CompilerParams(use_tc_tiling_on_sc=False),
)


def _worker_id():
    return lax.axis_index("s") * NC + lax.axis_index("c")


@_mesh(
    out_type=jax.ShapeDtypeStruct((NC, N_PAD), jnp.float32),
    scratch_types=[
        pltpu.VMEM((CHP, B), jnp.int32),
        pltpu.VMEM((B,), jnp.float32),
        pltpu.VMEM_SHARED((N_PAD,), jnp.float32),
        pltpu.SemaphoreType.DMA,
        pltpu.SemaphoreType.DMA,
    ],
)
def _sc_degree(ei2, zeros_hbm, ones_hbm, out, dst_v, ones_v, acc, semA, semB):
    c = lax.axis_index("c")
    sid = lax.axis_index("s")
    r0 = sid * RPS
    pltpu.sync_copy(zeros_hbm.at[pl.ds(r0, RPS)], acc.at[pl.ds(r0, RPS)])
    pltpu.sync_copy(ones_hbm, ones_v)
    pltpu.sync_copy(ei2.at[1, pl.ds(_worker_id() * CHP, CHP)], dst_v)
    plsc.subcore_barrier()

    def body(g, carry):
        for par, sem, oth in ((0, semA, semB), (1, semB, semA)):
            @pl.when(lax.rem(g, 2) == par)
            def _():
                for k in range(K):
                    pltpu.async_copy(
                        ones_v, acc.at[dst_v.at[g * K + k]], sem, add=True)

                @pl.when(g >= 1)
                def _():
                    for k in range(K):
                        pltpu.make_async_copy(
                            ones_v, acc.at[dst_v.at[(g - 1) * K + k]], oth,
                        ).wait()
        return carry

    lax.fori_loop(0, G_DEG, body, 0)
    lastsem = semB if (G_DEG - 1) % 2 else semA
    for k in range(K):
        pltpu.make_async_copy(
            ones_v, acc.at[dst_v.at[(G_DEG - 1) * K + k]], lastsem).wait()
    plsc.subcore_barrier()
    pltpu.sync_copy(acc.at[pl.ds(r0, RPS)], out.at[c, pl.ds(r0, RPS)])


@_mesh(
    out_type=jax.ShapeDtypeStruct((NC, N_PAD, D_HID), jnp.float32),
    scratch_types=[
        pltpu.VMEM((CHP, B), jnp.int32),
        pltpu.VMEM((CHP, B), jnp.int32),
        [pltpu.VMEM((B, D_HID), jnp.float32)] * (2 * K),
        pltpu.VMEM_SHARED((N_PAD, D_HID), jnp.float32),
        pltpu.SemaphoreType.DMA,
        pltpu.SemaphoreType.DMA,
        pltpu.SemaphoreType.DMA,
        pltpu.SemaphoreType.DMA,
    ],
)
def _sc_aggregate(ei2, g_hbm, zeros_hbm, out,
                  src_v, dst_v, rows, acc, gsemA, gsemB, ssemA, ssemB):
    c = lax.axis_index("c")
    sid = lax.axis_index("s")
    wid = _worker_id()
    r0 = sid * RPS
    pltpu.sync_copy(zeros_hbm.at[pl.ds(r0, RPS)], acc.at[pl.ds(r0, RPS)])
    pltpu.sync_copy(ei2.at[0, pl.ds(wid * CHP, CHP)], src_v)
    pltpu.sync_copy(ei2.at[1, pl.ds(wid * CHP, CHP)], dst_v)
    plsc.subcore_barrier()

    buf = (rows[:K], rows[K:])
    gsem = (gsemA, gsemB)
    ssem = (ssemA, ssemB)

    for k in range(K):
        pltpu.async_copy(g_hbm.at[src_v.at[k]], buf[0][k], gsem[0])

    def body(g, carry):
        for par in (0, 1):
            oth = 1 - par

            @pl.when(lax.rem(g, 2) == par)
            def _():
                @pl.when(g >= 1)
                def _():
                    for k in range(K):
                        pltpu.make_async_copy(
                            buf[oth][k],
                            acc.at[dst_v.at[(g - 1) * K + k]],
                            ssem[oth],
                        ).wait()

                @pl.when(g + 1 < G)
                def _():
                    for k in range(K):
                        pltpu.async_copy(
                            g_hbm.at[src_v.at[(g + 1) * K + k]],
                            buf[oth][k], gsem[oth])

                for k in range(K):
                    pltpu.make_async_copy(
                        g_hbm.at[src_v.at[g * K + k]],
                        buf[par][k], gsem[par],
                    ).wait()
                for k in range(K):
                    pltpu.async_copy(
                        buf[par][k], acc.at[dst_v.at[g * K + k]],
                        ssem[par], add=True)
        return carry

    lax.fori_loop(0, G, body, 0)
    lastpar = (G - 1) % 2
    for k in range(K):
        pltpu.make_async_copy(
            buf[lastpar][k], acc.at[dst_v.at[(G - 1) * K + k]],
            ssem[lastpar]).wait()
    plsc.subcore_barrier()
    pltpu.sync_copy(acc.at[pl.ds(r0, RPS)], out.at[c, pl.ds(r0, RPS)])


def _tc_h1_body(xp_ref, w1blk_ref, h1_ref):
    h1_ref[...] = jnp.dot(xp_ref[...], w1blk_ref[...],
                          preferred_element_type=jnp.float32)


def _tc_a_body(h1_ref, deg_ref, rrep_ref, pcat_ref, g1_ref, dinv_ref):
    deg1 = deg_ref[0] + deg_ref[1]
    u = jnp.dot(rrep_ref[...], deg1, preferred_element_type=jnp.float32)
    rowt = lax.rem(lax.broadcasted_iota(jnp.int32, (NR, 128), 0), 16)
    degp = jnp.zeros((NR, 128), jnp.float32)
    for t in range(16):
        yt = jnp.dot(u, pcat_ref[:, t * 128:(t + 1) * 128],
                     preferred_element_type=jnp.float32)
        degp = jnp.where(rowt == t, yt, degp)
    dinv = lax.rsqrt(1.0 + degp)
    dinv_ref[...] = dinv
    g1_ref[...] = dinv * h1_ref[...]


def _tc_b_body(s1_ref, g1_ref, dinv_ref, b1_ref, g2_ref):
    dinv = dinv_ref[...]
    s = s1_ref[0] + s1_ref[1] + g1_ref[...]
    g2_ref[...] = dinv * jnp.maximum(dinv * s + b1_ref[...], 0.0)


def _tc_c_body(s2_ref, g2_ref, dinv_ref, w2blk_ref, b2_ref, out_ref):
    a = dinv_ref[...] * (s2_ref[0] + s2_ref[1] + g2_ref[...])
    out_ref[...] = (
        jnp.dot(a, w2blk_ref[...], preferred_element_type=jnp.float32)
        + b2_ref[...]
    )


def kernel(x, edge_index, W1, b1, W2, b2):
    f32 = jnp.float32
    ei2 = edge_index.reshape(2, NW * CHP, B)
    zeros = jnp.zeros((N_PAD, D_HID), f32)
    zeros1 = jnp.zeros((N_PAD,), f32)
    ones1 = jnp.ones((B,), f32)

    rrep = jnp.kron(jnp.eye(80, dtype=f32), jnp.ones((16, 1), f32))
    eye8 = jnp.eye(8, dtype=f32)
    krep = jnp.kron(eye8, jnp.ones((1, 16), f32))
    pcat = jnp.concatenate(
        [jnp.kron(jnp.eye(16, dtype=f32)[:, t:t + 1], eye8) @ krep
         for t in range(16)], axis=1)

    xp = jnp.concatenate(
        [x, jnp.zeros((N_PAD - N, D_IN), f32)]).reshape(NR, 8 * D_IN)
    w1blk = jnp.kron(eye8, W1)
    w2blk = jnp.kron(eye8, W2)
    b1t = jnp.tile(b1, 8).reshape(1, 8 * D_HID)
    b2t = jnp.tile(b2, 8).reshape(1, 8 * D_OUT)

    h1 = pl.pallas_call(
        _tc_h1_body,
        out_shape=jax.ShapeDtypeStruct((NR, 128), f32),
    )(xp, w1blk)

    degacc = _sc_degree(ei2, zeros1, ones1).reshape(NC, 80, 128)

    g1, dinv = pl.pallas_call(
        _tc_a_body,
        out_shape=(
            jax.ShapeDtypeStruct((NR, 128), f32),
            jax.ShapeDtypeStruct((NR, 128), f32),
        ),
    )(h1, degacc, rrep, pcat)

    s1 = _sc_aggregate(ei2, g1.reshape(N_PAD, D_HID), zeros)

    g2 = pl.pallas_call(
        _tc_b_body,
        out_shape=jax.ShapeDtypeStruct((NR, 128), f32),
    )(s1.reshape(NC, NR, 128), g1, dinv, b1t)

    s2 = _sc_aggregate(ei2, g2.reshape(N_PAD, D_HID), zeros)

    outp = pl.pallas_call(
        _tc_c_body,
        out_shape=jax.ShapeDtypeStruct((NR, 8 * D_OUT), f32),
    )(s2.reshape(NC, NR, 128), g2, dinv, w2blk, b2t)

    return outp.reshape(N_PAD, D_OUT)[:N]


# speedup vs baseline: 1.0524x; 1.0524x over previous
import functools

import jax
import jax.numpy as jnp
from jax import lax
from jax.experimental import pallas as pl
from jax.experimental.pallas import tpu as pltpu
from jax.experimental.pallas import tpu_sc as plsc

N = 10000
E = 320000
D_IN = 128
D_HID = 16
D_OUT = 128

NC = 2
NS = 16
NW = NC * NS
B = 125
CHP = 80
N_PAD = 10240
RPS = N_PAD // NS
NR = N_PAD // 8
K = 8
G = CHP // K
G_DEG = G

_mesh = functools.partial(
    pl.kernel,
    mesh=plsc.VectorSubcoreMesh(core_axis_name="c", subcore_axis_name="s"),
    compiler_params=pltpu.CompilerParams(use_tc_tiling_on_sc=False),
)


def _worker_id():
    return lax.axis_index("s") * NC + lax.axis_index("c")


@_mesh(
    out_type=jax.ShapeDtypeStruct((NC, N_PAD), jnp.float32),
    scratch_types=[
        pltpu.VMEM((CHP, B), jnp.int32),
        pltpu.VMEM((B,), jnp.float32),
        pltpu.VMEM_SHARED((N_PAD,), jnp.float32),
        pltpu.SemaphoreType.DMA,
        pltpu.SemaphoreType.DMA,
    ],
)
def _sc_degree(ei2, zeros_hbm, ones_hbm, out, dst_v, ones_v, acc, semA, semB):
    c = lax.axis_index("c")
    sid = lax.axis_index("s")
    r0 = sid * RPS
    pltpu.sync_copy(zeros_hbm.at[pl.ds(r0, RPS)], acc.at[pl.ds(r0, RPS)])
    pltpu.sync_copy(ones_hbm, ones_v)
    pltpu.sync_copy(ei2.at[1, pl.ds(_worker_id() * CHP, CHP)], dst_v)
    plsc.subcore_barrier()

    def body(g, carry):
        for par, sem, oth in ((0, semA, semB), (1, semB, semA)):
            @pl.when(lax.rem(g, 2) == par)
            def _():
                for k in range(K):
                    pltpu.async_copy(
                        ones_v, acc.at[dst_v.at[g * K + k]], sem, add=True)

                @pl.when(g >= 1)
                def _():
                    for k in range(K):
                        pltpu.make_async_copy(
                            ones_v, acc.at[dst_v.at[(g - 1) * K + k]], oth,
                        ).wait()
        return carry

    lax.fori_loop(0, G_DEG, body, 0)
    lastsem = semB if (G_DEG - 1) % 2 else semA
    for k in range(K):
        pltpu.make_async_copy(
            ones_v, acc.at[dst_v.at[(G_DEG - 1) * K + k]], lastsem).wait()
    plsc.subcore_barrier()
    pltpu.sync_copy(acc.at[pl.ds(r0, RPS)], out.at[c, pl.ds(r0, RPS)])


@_mesh(
    out_type=jax.ShapeDtypeStruct((NC, N_PAD, D_HID), jnp.float32),
    scratch_types=[
        pltpu.VMEM((CHP, B), jnp.int32),
        pltpu.VMEM((CHP, B), jnp.int32),
        [pltpu.VMEM((B, D_HID), jnp.float32)] * (2 * K),
        pltpu.VMEM_SHARED((N_PAD, D_HID), jnp.float32),
        pltpu.SemaphoreType.DMA,
        pltpu.SemaphoreType.DMA,
        pltpu.SemaphoreType.DMA,
        pltpu.SemaphoreType.DMA,
    ],
)
def _sc_aggregate(ei2, g_hbm, zeros_hbm, out,
                  src_v, dst_v, rows, acc, gsemA, gsemB, ssemA, ssemB):
    c = lax.axis_index("c")
    sid = lax.axis_index("s")
    wid = _worker_id()
    r0 = sid * RPS
    pltpu.sync_copy(zeros_hbm.at[pl.ds(r0, RPS)], acc.at[pl.ds(r0, RPS)])
    pltpu.sync_copy(ei2.at[0, pl.ds(wid * CHP, CHP)], src_v)
    pltpu.sync_copy(ei2.at[1, pl.ds(wid * CHP, CHP)], dst_v)
    plsc.subcore_barrier()

    buf = (rows[:K], rows[K:])
    gsem = (gsemA, gsemB)
    ssem = (ssemA, ssemB)

    for k in range(K):
        pltpu.async_copy(g_hbm.at[src_v.at[k]], buf[0][k], gsem[0])

    def body(g, carry):
        for par in (0, 1):
            oth = 1 - par

            @pl.when(lax.rem(g, 2) == par)
            def _():
                @pl.when(g >= 1)
                def _():
                    for k in range(K):
                        pltpu.make_async_copy(
                            buf[oth][k],
                            acc.at[dst_v.at[(g - 1) * K + k]],
                            ssem[oth],
                        ).wait()

                @pl.when(g + 1 < G)
                def _():
                    for k in range(K):
                        pltpu.async_copy(
                            g_hbm.at[src_v.at[(g + 1) * K + k]],
                            buf[oth][k], gsem[oth])

                for k in range(K):
                    pltpu.make_async_copy(
                        g_hbm.at[src_v.at[g * K + k]],
                        buf[par][k], gsem[par],
                    ).wait()
                for k in range(K):
                    pltpu.async_copy(
                        buf[par][k], acc.at[dst_v.at[g * K + k]],
                        ssem[par], add=True)
        return carry

    lax.fori_loop(0, G, body, 0)
    lastpar = (G - 1) % 2
    for k in range(K):
        pltpu.make_async_copy(
            buf[lastpar][k], acc.at[dst_v.at[(G - 1) * K + k]],
            ssem[lastpar]).wait()
    plsc.subcore_barrier()
    pltpu.sync_copy(acc.at[pl.ds(r0, RPS)], out.at[c, pl.ds(r0, RPS)])


def _tc_h1_body(xp_ref, w1blk_ref, h1_ref):
    h1_ref[...] = jnp.dot(xp_ref[...], w1blk_ref[...],
                          preferred_element_type=jnp.float32)


def _tc_a_body(h1_ref, deg_ref, g1_ref, dinv_ref):
    f32 = jnp.float32
    deg1 = deg_ref[0] + deg_ref[1]
    rr = lax.broadcasted_iota(jnp.int32, (NR, 80), 0) >> 4
    rq = lax.broadcasted_iota(jnp.int32, (NR, 80), 1)
    rrep = (rr == rq).astype(f32)
    u = jnp.dot(rrep, deg1, preferred_element_type=f32)
    pl_ = lax.broadcasted_iota(jnp.int32, (128, 128), 0)
    pk = lax.broadcasted_iota(jnp.int32, (128, 128), 1) >> 4
    rowt = lax.rem(lax.broadcasted_iota(jnp.int32, (NR, 128), 0), 16)
    degp = jnp.zeros((NR, 128), f32)
    for t in range(16):
        pt = (pl_ == 8 * t + pk).astype(f32)
        yt = jnp.dot(u, pt, preferred_element_type=f32)
        degp = jnp.where(rowt == t, yt, degp)
    dinv = lax.rsqrt(1.0 + degp)
    dinv_ref[...] = dinv
    g1_ref[...] = dinv * h1_ref[...]


def _tc_b_body(s1_ref, g1_ref, dinv_ref, b1_ref, g2_ref):
    dinv = dinv_ref[...]
    s = s1_ref[0] + s1_ref[1] + g1_ref[...]
    g2_ref[...] = dinv * jnp.maximum(dinv * s + b1_ref[...], 0.0)


def _tc_c_body(s2_ref, g2_ref, dinv_ref, w2blk_ref, b2_ref, out_ref):
    a = dinv_ref[...] * (s2_ref[0] + s2_ref[1] + g2_ref[...])
    out_ref[...] = (
        jnp.dot(a, w2blk_ref[...], preferred_element_type=jnp.float32)
        + b2_ref[...]
    )


def kernel(x, edge_index, W1, b1, W2, b2):
    f32 = jnp.float32
    ei2 = edge_index.reshape(2, NW * CHP, B)
    zeros = jnp.zeros((N_PAD, D_HID), f32)
    zeros1 = jnp.zeros((N_PAD,), f32)
    ones1 = jnp.ones((B,), f32)
    eye8 = jnp.eye(8, dtype=f32)

    xp = jnp.concatenate(
        [x, jnp.zeros((N_PAD - N, D_IN), f32)]).reshape(NR, 8 * D_IN)
    w1blk = jnp.kron(eye8, W1)
    w2blk = jnp.kron(eye8, W2)
    b1t = jnp.tile(b1, 8).reshape(1, 8 * D_HID)
    b2t = jnp.tile(b2, 8).reshape(1, 8 * D_OUT)

    h1 = pl.pallas_call(
        _tc_h1_body,
        out_shape=jax.ShapeDtypeStruct((NR, 128), f32),
    )(xp, w1blk)

    degacc = _sc_degree(ei2, zeros1, ones1).reshape(NC, 80, 128)

    g1, dinv = pl.pallas_call(
        _tc_a_body,
        out_shape=(
            jax.ShapeDtypeStruct((NR, 128), f32),
            jax.ShapeDtypeStruct((NR, 128), f32),
        ),
    )(h1, degacc)

    s1 = _sc_aggregate(ei2, g1.reshape(N_PAD, D_HID), zeros)

    g2 = pl.pallas_call(
        _tc_b_body,
        out_shape=jax.ShapeDtypeStruct((NR, 128), f32),
    )(s1.reshape(NC, NR, 128), g1, dinv, b1t)

    s2 = _sc_aggregate(ei2, g2.reshape(N_PAD, D_HID), zeros)

    outp = pl.pallas_call(
        _tc_c_body,
        out_shape=jax.ShapeDtypeStruct((NR, 8 * D_OUT), f32),
    )(s2.reshape(NC, NR, 128), g2, dinv, w2blk, b2t)

    return outp.reshape(N_PAD, D_OUT)[:N]
